# baseline (speedup 1.0000x reference)
import functools

import jax
import jax.numpy as jnp
from jax import lax
from jax.experimental import pallas as pl
from jax.experimental.pallas import tpu as pltpu
from jax.experimental.pallas import tpu_sc as plsc

_B, _T, _C = 8, 4096, 64
_P = 4
_D = 256
_K = 1024
_N = (_B * _T) // _P
_BLK = 512
_GRID = _N // _BLK

_NC, _NS = 2, 16
_NW = _NC * _NS
_RPW = _N // _NW
_CH = 128
_NCH = _RPW // _CH


def _vq_codes_body(x_ref, we_ref, be_ref, cb_ref, wd_ref, bd_ref,
                   codes_ref, dec_ref, e2_ref):
    i = pl.program_id(0)

    @pl.when(i == 0)
    def _():
        cb = cb_ref[...]
        dec_ref[...] = (
            jnp.dot(cb, wd_ref[...], preferred_element_type=jnp.float32)
            + bd_ref[...]
        )
        e2_ref[...] = jnp.sum(cb * cb, axis=1, keepdims=True).T

    x = x_ref[...]
    z = jnp.dot(x, we_ref[...], preferred_element_type=jnp.float32) + be_ref[...]
    s = lax.dot_general(
        z, cb_ref[...], (((1,), (1,)), ((), ())),
        preferred_element_type=jnp.float32)
    z2 = jnp.sum(z * z, axis=1, keepdims=True)
    dists = z2 - 2.0 * s + e2_ref[...]
    minval = jnp.min(dists, axis=1, keepdims=True)
    iota = lax.broadcasted_iota(jnp.int32, (_BLK, _K), 1)
    idx = jnp.min(jnp.where(dists == minval, iota, _K), axis=1)
    codes_ref[...] = idx.reshape(1, 1, _BLK)


def _tc_codes(x, W_enc, b_enc, codebook, W_dec, b_dec):
    return pl.pallas_call(
        _vq_codes_body,
        grid=(_GRID,),
        in_specs=[
            pl.BlockSpec((_BLK, _P * _C), lambda i: (i, 0)),
            pl.BlockSpec((_P * _C, _D), lambda i: (0, 0)),
            pl.BlockSpec((1, _D), lambda i: (0, 0)),
            pl.BlockSpec((_K, _D), lambda i: (0, 0)),
            pl.BlockSpec((_D, _P * _C), lambda i: (0, 0)),
            pl.BlockSpec((1, _P * _C), lambda i: (0, 0)),
        ],
        out_specs=[
            pl.BlockSpec((1, 1, _BLK), lambda i: (i, 0, 0)),
            pl.BlockSpec((_K, _P * _C), lambda i: (0, 0)),
        ],
        out_shape=[
            jax.ShapeDtypeStruct((_GRID, 1, _BLK), jnp.int32),
            jax.ShapeDtypeStruct((_K, _P * _C), jnp.float32),
        ],
        scratch_shapes=[pltpu.VMEM((1, _K), jnp.float32)],
    )(x, W_enc, b_enc.reshape(1, _D), codebook, W_dec,
      b_dec.reshape(1, _P * _C))


def _sc_gather_body(codes_hbm, table_hbm, out_hbm, idx_v, rows_v, sem):
    wid = lax.axis_index("s") * _NC + lax.axis_index("c")
    pltpu.sync_copy(codes_hbm.at[pl.ds(wid * _NCH, _NCH)], idx_v)
    copies = [
        pltpu.async_copy(table_hbm.at[idx_v.at[j]], rows_v.at[j], sem)
        for j in range(_NCH)
    ]
    for c in copies:
        c.wait()
    base = wid * _RPW
    for j in range(_NCH):
        pltpu.sync_copy(rows_v.at[j], out_hbm.at[pl.ds(base + j * _CH, _CH)])


_sc_gather = functools.partial(
    pl.kernel,
    mesh=plsc.VectorSubcoreMesh(core_axis_name="c", subcore_axis_name="s"),
    out_type=jax.ShapeDtypeStruct((_N, _P * _C), jnp.float32),
    scratch_types=[
        pltpu.VMEM((_NCH, _CH), jnp.int32),
        pltpu.VMEM((_NCH, _CH, _P * _C), jnp.float32),
        pltpu.SemaphoreType.DMA,
    ],
)(_sc_gather_body)


def kernel(brain_wave, W_enc, b_enc, codebook, W_dec, b_dec):
    x = brain_wave.reshape(_N, _P * _C)
    codes3, dec_cb = _tc_codes(x, W_enc, b_enc, codebook, W_dec, b_dec)
    codes = codes3.reshape(_N // _CH, _CH)
    out = _sc_gather(codes, dec_cb)
    return out.reshape(_B, _T, _C)


# speedup vs baseline: 1.1283x; 1.1283x over previous
import functools

import jax
import jax.numpy as jnp
from jax import lax
from jax.experimental import pallas as pl
from jax.experimental.pallas import tpu as pltpu
from jax.experimental.pallas import tpu_sc as plsc

_B, _T, _C = 8, 4096, 64
_P = 4
_D = 256
_K = 1024
_N = (_B * _T) // _P
_BLK = 1024
_GRID = _N // _BLK

_NC, _NS = 2, 16
_NW = _NC * _NS
_RPW = _N // _NW
_CH = 64
_NCH = _RPW // _CH


def _vq_codes_body(x_ref, we_ref, be_ref, cb_ref, wd_ref, bd_ref,
                   codes_ref, dec_ref, e2_ref):
    i = pl.program_id(0)

    @pl.when(i == 0)
    def _():
        cb = cb_ref[...]
        dec_ref[...] = (
            jnp.dot(cb, wd_ref[...], preferred_element_type=jnp.float32)
            + bd_ref[...]
        )
        e2_ref[...] = jnp.sum(cb * cb, axis=1, keepdims=True).T

    x = x_ref[...]
    z = jnp.dot(x, we_ref[...], preferred_element_type=jnp.float32) + be_ref[...]
    s = lax.dot_general(
        z, cb_ref[...], (((1,), (1,)), ((), ())),
        preferred_element_type=jnp.float32)
    z2 = jnp.sum(z * z, axis=1, keepdims=True)
    dists = z2 - 2.0 * s + e2_ref[...]
    minval = jnp.min(dists, axis=1, keepdims=True)
    iota = lax.broadcasted_iota(jnp.int32, (_BLK, _K), 1)
    idx = jnp.min(jnp.where(dists == minval, iota, _K), axis=1)
    codes_ref[...] = idx.reshape(_BLK // _CH, _CH)


def _tc_codes(x, W_enc, b_enc, codebook, W_dec, b_dec):
    return pl.pallas_call(
        _vq_codes_body,
        grid=(_GRID,),
        in_specs=[
            pl.BlockSpec((_BLK, _P * _C), lambda i: (i, 0)),
            pl.BlockSpec((_P * _C, _D), lambda i: (0, 0)),
            pl.BlockSpec((1, _D), lambda i: (0, 0)),
            pl.BlockSpec((_K, _D), lambda i: (0, 0)),
            pl.BlockSpec((_D, _P * _C), lambda i: (0, 0)),
            pl.BlockSpec((1, _P * _C), lambda i: (0, 0)),
        ],
        out_specs=[
            pl.BlockSpec((_BLK // _CH, _CH), lambda i: (i, 0)),
            pl.BlockSpec((_K, _P * _C), lambda i: (0, 0)),
        ],
        out_shape=[
            jax.ShapeDtypeStruct((_N // _CH, _CH), jnp.int32),
            jax.ShapeDtypeStruct((_K, _P * _C), jnp.float32),
        ],
        scratch_shapes=[pltpu.VMEM((1, _K), jnp.float32)],
    )(x, W_enc, b_enc.reshape(1, _D), codebook, W_dec,
      b_dec.reshape(1, _P * _C))


def _sc_gather_body(codes_hbm, table_hbm, out_hbm, idx_v, rows_v,
                    g0, g1, g2, g3, sem_w):
    wid = lax.axis_index("s") * _NC + lax.axis_index("c")
    pltpu.sync_copy(codes_hbm.at[pl.ds(wid * _NCH, _NCH)], idx_v)
    base = wid * _RPW
    gsems = [g0, g1, g2, g3]
    gathers = [
        pltpu.async_copy(table_hbm.at[idx_v.at[j]], rows_v.at[j], gsems[j])
        for j in range(_NCH)
    ]
    writes = []
    for j in range(_NCH):
        gathers[j].wait()
        writes.append(
            pltpu.async_copy(rows_v.at[j],
                             out_hbm.at[pl.ds(base + j * _CH, _CH)], sem_w))
    for w in writes:
        w.wait()


_sc_gather = functools.partial(
    pl.kernel,
    mesh=plsc.VectorSubcoreMesh(core_axis_name="c", subcore_axis_name="s"),
    out_type=jax.ShapeDtypeStruct((_N, _P * _C), jnp.float32),
    scratch_types=[
        pltpu.VMEM((_NCH, _CH), jnp.int32),
        pltpu.VMEM((_NCH, _CH, _P * _C), jnp.float32),
        pltpu.SemaphoreType.DMA,
        pltpu.SemaphoreType.DMA,
        pltpu.SemaphoreType.DMA,
        pltpu.SemaphoreType.DMA,
        pltpu.SemaphoreType.DMA,
    ],
)(_sc_gather_body)


def kernel(brain_wave, W_enc, b_enc, codebook, W_dec, b_dec):
    x = brain_wave.reshape(_N, _P * _C)
    codes, dec_cb = _tc_codes(x, W_enc, b_enc, codebook, W_dec, b_dec)
    out = _sc_gather(codes, dec_cb)
    return out.reshape(_B, _T, _C)


# speedup vs baseline: 1.2360x; 1.0954x over previous
import functools

import jax
import jax.numpy as jnp
from jax import lax
from jax.experimental import pallas as pl
from jax.experimental.pallas import tpu as pltpu
from jax.experimental.pallas import tpu_sc as plsc

_B, _T, _C = 8, 4096, 64
_P = 4
_D = 256
_K = 1024
_N = (_B * _T) // _P
_BLK = 1024
_GRID = _N // _BLK
_H = 2

_NC, _NS = 2, 16
_NW = _NC * _NS
_EPW = _N * _H // _NW
_GCH = 128
_NG = _EPW // _GCH


def _vq_codes_body(x_ref, we_ref, be_ref, cb_ref, wd_ref, bd_ref,
                   idx2_ref, dec_ref, e2_ref):
    i = pl.program_id(0)

    @pl.when(i == 0)
    def _():
        cb = cb_ref[...]
        for h in range(_H):
            dec_ref[h] = (
                jnp.dot(cb, wd_ref[:, h * 128:(h + 1) * 128],
                        preferred_element_type=jnp.float32)
                + bd_ref[:, h * 128:(h + 1) * 128]
            )
        e2_ref[...] = jnp.sum(cb * cb, axis=1, keepdims=True).T

    x = jnp.concatenate(
        [x_ref[pl.Slice(r, _BLK, _P), :] for r in range(_P)], axis=1)
    z = jnp.dot(x, we_ref[...], preferred_element_type=jnp.float32) + be_ref[...]
    s = lax.dot_general(
        z, cb_ref[...], (((1,), (1,)), ((), ())),
        preferred_element_type=jnp.float32)
    z2 = jnp.sum(z * z, axis=1, keepdims=True)
    dists = z2 - 2.0 * s + e2_ref[...]
    minval = jnp.min(dists, axis=1, keepdims=True)
    iota = lax.broadcasted_iota(jnp.int32, (_BLK, _K), 1)
    idx = jnp.min(jnp.where(dists == minval, iota, _K), axis=1)
    rows = _BLK * _H // 128
    codes64 = idx.reshape(rows, 128 // _H).astype(jnp.float32)
    rsel = (lax.broadcasted_iota(jnp.int32, (128 // _H, 128), 0)
            == lax.broadcasted_iota(jnp.int32, (128 // _H, 128), 1) // _H
            ).astype(jnp.float32)
    rep = lax.dot_general(codes64, rsel, (((1,), (0,)), ((), ())),
                          precision=lax.Precision.HIGHEST,
                          preferred_element_type=jnp.float32)
    half = lax.bitwise_and(
        lax.broadcasted_iota(jnp.int32, (rows, 128), 1), _H - 1)
    idx2_ref[...] = lax.bitwise_or(rep.astype(jnp.int32),
                                   lax.shift_left(half, 10))


def _tc_codes(x, W_enc, b_enc, codebook, W_dec, b_dec):
    return pl.pallas_call(
        _vq_codes_body,
        grid=(_GRID,),
        in_specs=[
            pl.BlockSpec((_BLK * _P, _C), lambda i: (i, 0)),
            pl.BlockSpec((_P * _C, _D), lambda i: (0, 0)),
            pl.BlockSpec((1, _D), lambda i: (0, 0)),
            pl.BlockSpec((_K, _D), lambda i: (0, 0)),
            pl.BlockSpec((_D, _P * _C), lambda i: (0, 0)),
            pl.BlockSpec((1, _P * _C), lambda i: (0, 0)),
        ],
        out_specs=[
            pl.BlockSpec((_BLK * _H // 128, 128), lambda i: (i, 0)),
            pl.BlockSpec((_H, _K, 128), lambda i: (0, 0, 0)),
        ],
        out_shape=[
            jax.ShapeDtypeStruct((_N * _H // 128, 128), jnp.int32),
            jax.ShapeDtypeStruct((_H, _K, 128), jnp.float32),
        ],
        scratch_shapes=[pltpu.VMEM((1, _K), jnp.float32)],
    )(x, W_enc, b_enc.reshape(1, _D), codebook, W_dec,
      b_dec.reshape(1, _P * _C))


def _sc_gather_body(idx2_hbm, table_hbm, out_hbm, idx_v, rows_v,
                    g0, g1, g2, g3, sem_w):
    wid = lax.axis_index("s") * _NC + lax.axis_index("c")
    pltpu.sync_copy(idx2_hbm.at[pl.ds(wid * _NG, _NG)], idx_v)
    gsems = [g0, g1, g2, g3]
    gathers = [
        pltpu.async_copy(table_hbm.at[idx_v.at[j]], rows_v.at[j], gsems[j])
        for j in range(_NG)
    ]
    base = wid * _EPW
    writes = []
    for j in range(_NG):
        gathers[j].wait()
        writes.append(
            pltpu.async_copy(rows_v.at[j],
                             out_hbm.at[pl.ds(base + j * _GCH, _GCH)],
                             sem_w))
    for w in writes:
        w.wait()


_sc_gather = functools.partial(
    pl.kernel,
    mesh=plsc.VectorSubcoreMesh(core_axis_name="c", subcore_axis_name="s"),
    out_type=jax.ShapeDtypeStruct((_N * _H, 128), jnp.float32),
    scratch_types=[
        pltpu.VMEM((_NG, _GCH), jnp.int32),
        pltpu.VMEM((_NG, _GCH, 128), jnp.float32),
        pltpu.SemaphoreType.DMA,
        pltpu.SemaphoreType.DMA,
        pltpu.SemaphoreType.DMA,
        pltpu.SemaphoreType.DMA,
        pltpu.SemaphoreType.DMA,
    ],
)(_sc_gather_body)


def kernel(brain_wave, W_enc, b_enc, codebook, W_dec, b_dec):
    x = brain_wave.reshape(_N * _P, _C)
    idx2, dec2 = _tc_codes(x, W_enc, b_enc, codebook, W_dec, b_dec)
    table = dec2.reshape(_H * _K, 128)
    out = _sc_gather(idx2, table)
    return out.reshape(_B, _T, _C)


# speedup vs baseline: 1.5824x; 1.2802x over previous
import jax
import jax.numpy as jnp
from jax import lax
from jax.experimental import pallas as pl
from jax.experimental.pallas import tpu as pltpu

_B, _T, _C = 8, 4096, 64
_P = 4
_D = 256
_K = 1024
_N = (_B * _T) // _P
_BLK = 1024
_GRID = _N // _BLK


def _vq_body(x_ref, we_ref, be_ref, cb_ref, wd_ref, bd_ref, out_ref,
             dec0, dec1, dec2, dec3, e2_ref):
    i = pl.program_id(0)
    decs = [dec0, dec1, dec2, dec3]

    @pl.when(i == 0)
    def _():
        cb = cb_ref[...]
        for r in range(_P):
            decs[r][...] = (
                jnp.dot(cb, wd_ref[:, r * _C:(r + 1) * _C],
                        preferred_element_type=jnp.float32)
                + bd_ref[:, r * _C:(r + 1) * _C]
            )
        e2_ref[...] = jnp.sum(cb * cb, axis=1, keepdims=True).T

    x = jnp.concatenate(
        [x_ref[pl.Slice(r, _BLK, _P), :] for r in range(_P)], axis=1)
    z = jnp.dot(x, we_ref[...], preferred_element_type=jnp.float32) + be_ref[...]
    s = lax.dot_general(
        z, cb_ref[...], (((1,), (1,)), ((), ())),
        preferred_element_type=jnp.float32)
    z2 = jnp.sum(z * z, axis=1, keepdims=True)
    dists = z2 - 2.0 * s + e2_ref[...]
    minval = jnp.min(dists, axis=1, keepdims=True)
    iota = lax.broadcasted_iota(jnp.int32, (_BLK, _K), 1)
    idx = jnp.min(jnp.where(dists == minval, iota, _K), axis=1)
    one_hot = (iota == idx[:, None]).astype(jnp.float32)
    for r in range(_P):
        q_r = jnp.dot(one_hot, decs[r][...],
                      preferred_element_type=jnp.float32)
        out_ref[0, pl.Slice(r, _BLK, _P), :] = q_r


def kernel(brain_wave, W_enc, b_enc, codebook, W_dec, b_dec):
    x = brain_wave.reshape(_N * _P, _C)
    out = pl.pallas_call(
        _vq_body,
        grid=(_GRID,),
        in_specs=[
            pl.BlockSpec((_BLK * _P, _C), lambda i: (i, 0)),
            pl.BlockSpec((_P * _C, _D), lambda i: (0, 0)),
            pl.BlockSpec((1, _D), lambda i: (0, 0)),
            pl.BlockSpec((_K, _D), lambda i: (0, 0)),
            pl.BlockSpec((_D, _P * _C), lambda i: (0, 0)),
            pl.BlockSpec((1, _P * _C), lambda i: (0, 0)),
        ],
        out_specs=pl.BlockSpec((1, _T, _C), lambda i: (i, 0, 0)),
        out_shape=jax.ShapeDtypeStruct((_B, _T, _C), jnp.float32),
        scratch_shapes=[
            pltpu.VMEM((_K, _C), jnp.float32),
            pltpu.VMEM((_K, _C), jnp.float32),
            pltpu.VMEM((_K, _C), jnp.float32),
            pltpu.VMEM((_K, _C), jnp.float32),
            pltpu.VMEM((1, _K), jnp.float32),
        ],
    )(x, W_enc, b_enc.reshape(1, _D), codebook, W_dec,
      b_dec.reshape(1, _P * _C))
    return out


# speedup vs baseline: 1.9288x; 1.2189x over previous
import jax
import jax.numpy as jnp
from jax import lax
from jax.experimental import pallas as pl
from jax.experimental.pallas import tpu as pltpu

_B, _T, _C = 8, 4096, 64
_P = 4
_D = 256
_K = 1024
_N = (_B * _T) // _P
_BLK = 1024
_GRID = _N // _BLK


def _vq_body(x_ref, we_ref, be_ref, cb_ref, wd_ref, bd_ref, out_ref,
             dec0, dec1, dec2, dec3, e2_ref):
    i = pl.program_id(0)
    decs = [dec0, dec1, dec2, dec3]

    @pl.when(i == 0)
    def _():
        cb = cb_ref[...]
        for r in range(_P):
            decs[r][...] = (
                jnp.dot(cb, wd_ref[:, r * _C:(r + 1) * _C],
                        preferred_element_type=jnp.float32)
                + bd_ref[:, r * _C:(r + 1) * _C]
            )
        e2_ref[...] = jnp.sum(cb * cb, axis=1, keepdims=True).T

    x = jnp.concatenate(
        [x_ref[0, pl.Slice(r, _BLK, _P), :] for r in range(_P)], axis=1)
    z = jnp.dot(x, we_ref[...], preferred_element_type=jnp.float32) + be_ref[...]
    s = lax.dot_general(
        z, cb_ref[...], (((1,), (1,)), ((), ())),
        preferred_element_type=jnp.float32)
    z2 = jnp.sum(z * z, axis=1, keepdims=True)
    dists = z2 - 2.0 * s + e2_ref[...]
    minval = jnp.min(dists, axis=1, keepdims=True)
    iota = lax.broadcasted_iota(jnp.int32, (_BLK, _K), 1)
    idx = jnp.min(jnp.where(dists == minval, iota, _K), axis=1)
    one_hot = (iota == idx[:, None]).astype(jnp.float32)
    for r in range(_P):
        q_r = jnp.dot(one_hot, decs[r][...],
                      preferred_element_type=jnp.float32)
        out_ref[0, pl.Slice(r, _BLK, _P), :] = q_r


def kernel(brain_wave, W_enc, b_enc, codebook, W_dec, b_dec):
    x = brain_wave
    out = pl.pallas_call(
        _vq_body,
        grid=(_GRID,),
        in_specs=[
            pl.BlockSpec((1, _T, _C), lambda i: (i, 0, 0)),
            pl.BlockSpec((_P * _C, _D), lambda i: (0, 0)),
            pl.BlockSpec((1, _D), lambda i: (0, 0)),
            pl.BlockSpec((_K, _D), lambda i: (0, 0)),
            pl.BlockSpec((_D, _P * _C), lambda i: (0, 0)),
            pl.BlockSpec((1, _P * _C), lambda i: (0, 0)),
        ],
        out_specs=pl.BlockSpec((1, _T, _C), lambda i: (i, 0, 0)),
        out_shape=jax.ShapeDtypeStruct((_B, _T, _C), jnp.float32),
        scratch_shapes=[
            pltpu.VMEM((_K, _C), jnp.float32),
            pltpu.VMEM((_K, _C), jnp.float32),
            pltpu.VMEM((_K, _C), jnp.float32),
            pltpu.VMEM((_K, _C), jnp.float32),
            pltpu.VMEM((1, _K), jnp.float32),
        ],
    )(x, W_enc, b_enc.reshape(1, _D), codebook, W_dec,
      b_dec.reshape(1, _P * _C))
    return out
